# Initial kernel scaffold; baseline (speedup 1.0000x reference)
import jax
import jax.numpy as jnp
from jax.experimental import pallas as pl


def kernel(pairwise_g, coset_functions, mask, loc_W1, loc_b1, loc_W2, loc_b2, loc_W3, loc_b3, Wq, bq, Wk, bk, W_in, b_in, W_out, b_out):
    raise NotImplementedError("write your pallas kernel here")


# speedup vs baseline: 373.7810x; 373.7810x over previous
import functools

import jax
import jax.numpy as jnp
from jax.experimental import pallas as pl
from jax.experimental.pallas import tpu as pltpu

BN = 128


def _proj_kernel(coset_ref, Wq_ref, bq_ref, Wk_ref, bk_ref, Wv_ref, bv_ref,
                 q_ref, kT_ref, v_ref, *, scale):
    x = coset_ref[0]
    q = jax.lax.dot(x, Wq_ref[...], preferred_element_type=jnp.float32)
    q_ref[0] = (q + bq_ref[...]) * scale
    k = jax.lax.dot(x, Wk_ref[...], preferred_element_type=jnp.float32)
    kT_ref[0] = (k + bk_ref[...]).T
    v = jax.lax.dot(x, Wv_ref[...], preferred_element_type=jnp.float32)
    v_ref[0] = v + bv_ref[...]


def _main_kernel(pg_ref, q_ref, kT_ref, v_ref, mb_ref,
                 W1_ref, b1_ref, W2_ref, b2_ref, W3_ref, b3_ref,
                 Wo_ref, bo_ref, out_ref, *, gd, hid, nh, dh, dv):
    f32 = jnp.float32

    def swish(a):
        return a * jax.nn.sigmoid(a)

    xs = [pg_ref[0, i] for i in range(gd)]
    h1 = []
    for j in range(hid):
        acc = xs[0] * W1_ref[0, j] + b1_ref[j]
        for i in range(1, gd):
            acc += xs[i] * W1_ref[i, j]
        h1.append(swish(acc))
    h2 = []
    for j in range(hid):
        acc = h1[0] * W2_ref[0, j] + b2_ref[j]
        for i in range(1, hid):
            acc += h1[i] * W2_ref[i, j]
        h2.append(swish(acc))

    qb = q_ref[0]
    mb = mb_ref[0]
    outs = []
    for h in range(nh):
        loc = h2[0] * W3_ref[0, h] + b3_ref[h]
        for i in range(1, hid):
            loc += h2[i] * W3_ref[i, h]
        dots = jax.lax.dot(qb[:, h * dh:(h + 1) * dh],
                           kT_ref[0, h * dh:(h + 1) * dh, :],
                           preferred_element_type=f32)
        e = jnp.exp(loc + dots + mb)
        s = jnp.sum(e, axis=-1, keepdims=True)
        ov = jax.lax.dot(e, v_ref[0, :, h * dv:(h + 1) * dv],
                         preferred_element_type=f32)
        outs.append(ov / s)
    o = jnp.concatenate(outs, axis=-1)
    out_ref[0] = jax.lax.dot(o, Wo_ref[...],
                             preferred_element_type=f32) + bo_ref[...]


def kernel(pairwise_g, coset_functions, mask, loc_W1, loc_b1, loc_W2, loc_b2,
           loc_W3, loc_b3, Wq, bq, Wk, bk, W_in, b_in, W_out, b_out):
    bs, n, d = coset_functions.shape
    gd = pairwise_g.shape[-1]
    hid = loc_b1.shape[0]
    nh = loc_b3.shape[0]
    dh = d // nh
    c_out = b_in.shape[0]
    dv = c_out // nh
    f32 = jnp.float32

    pg_t = jnp.transpose(pairwise_g, (0, 3, 1, 2))
    mask_bias = jnp.where(mask, 0.0, -1e38).astype(f32).reshape(bs, 1, n)

    proj = pl.pallas_call(
        functools.partial(_proj_kernel, scale=1.0 / (dh ** 0.5)),
        grid=(bs,),
        in_specs=[
            pl.BlockSpec((1, n, d), lambda b: (b, 0, 0)),
            pl.BlockSpec((d, d), lambda b: (0, 0)),
            pl.BlockSpec((1, d), lambda b: (0, 0)),
            pl.BlockSpec((d, d), lambda b: (0, 0)),
            pl.BlockSpec((1, d), lambda b: (0, 0)),
            pl.BlockSpec((d, c_out), lambda b: (0, 0)),
            pl.BlockSpec((1, c_out), lambda b: (0, 0)),
        ],
        out_specs=[
            pl.BlockSpec((1, n, d), lambda b: (b, 0, 0)),
            pl.BlockSpec((1, d, n), lambda b: (b, 0, 0)),
            pl.BlockSpec((1, n, c_out), lambda b: (b, 0, 0)),
        ],
        out_shape=[
            jax.ShapeDtypeStruct((bs, n, d), f32),
            jax.ShapeDtypeStruct((bs, d, n), f32),
            jax.ShapeDtypeStruct((bs, n, c_out), f32),
        ],
    )
    q, kT, v = proj(coset_functions, Wq, bq.reshape(1, d), Wk,
                    bk.reshape(1, d), W_in, b_in.reshape(1, c_out))

    smem = pl.BlockSpec(memory_space=pltpu.SMEM)
    out = pl.pallas_call(
        functools.partial(_main_kernel, gd=gd, hid=hid, nh=nh, dh=dh, dv=dv),
        grid=(bs, n // BN),
        in_specs=[
            pl.BlockSpec((1, gd, BN, n), lambda b, i: (b, 0, i, 0)),
            pl.BlockSpec((1, BN, d), lambda b, i: (b, i, 0)),
            pl.BlockSpec((1, d, n), lambda b, i: (b, 0, 0)),
            pl.BlockSpec((1, n, c_out), lambda b, i: (b, 0, 0)),
            pl.BlockSpec((1, 1, n), lambda b, i: (b, 0, 0)),
            smem, smem, smem, smem, smem, smem,
            pl.BlockSpec((c_out, c_out), lambda b, i: (0, 0)),
            pl.BlockSpec((1, c_out), lambda b, i: (0, 0)),
        ],
        out_specs=pl.BlockSpec((1, BN, c_out), lambda b, i: (b, i, 0)),
        out_shape=jax.ShapeDtypeStruct((bs, n, c_out), f32),
        compiler_params=pltpu.CompilerParams(
            dimension_semantics=("parallel", "parallel")),
    )(pg_t, q, kT, v, mask_bias,
      loc_W1, loc_b1, loc_W2, loc_b2, loc_W3, loc_b3,
      W_out, b_out.reshape(1, c_out))

    return (pairwise_g, out, mask)
